# baseline (speedup 1.0000x reference)
import functools

import jax
import jax.numpy as jnp
from jax import lax
from jax.experimental import pallas as pl
from jax.experimental.pallas import tpu as pltpu
from jax.experimental.pallas import tpu_sc as plsc

N = 10000
E = 320000
IN = 128
H = 64
CH = 512
NCLS = 2
SLOPE = 11.0 / 48.0

NP = 10240
NTILES = 32
CHK = 128
CPT = 80
EPAD = NTILES * CPT * CHK
RPT = NP // 16


def _mesh():
    return plsc.VectorSubcoreMesh(core_axis_name="c", subcore_axis_name="s")


def _sc_degrees(src_r, dst_r):
    @functools.partial(
        pl.kernel,
        out_type=[jax.ShapeDtypeStruct((2, NP, 16), jnp.float32),
                  jax.ShapeDtypeStruct((2, NP, 16), jnp.float32)],
        mesh=_mesh(),
        scratch_types=[
            pltpu.VMEM((CPT, CHK), jnp.int32),
            pltpu.VMEM((CPT, CHK), jnp.int32),
            pltpu.VMEM((CHK, 16), jnp.float32),
            pltpu.VMEM((CHK, 16), jnp.float32),
            pltpu.VMEM_SHARED((NP, 16), jnp.float32),
            pltpu.VMEM_SHARED((NP, 16), jnp.float32),
        ],
        compiler_params=pltpu.CompilerParams(use_tc_tiling_on_sc=False),
    )
    def deg_kernel(src_hbm, dst_hbm, outd_hbm, ind_hbm,
                   srcv, dstv, onesv, zv, outa, ina):
        c = lax.axis_index("c")
        s = lax.axis_index("s")
        w = c * 16 + s

        def fill(i, carry):
            onesv[i, :] = jnp.ones((16,), jnp.float32)
            zv[i, :] = jnp.zeros((16,), jnp.float32)
            return carry
        lax.fori_loop(0, CHK, fill, 0)

        base = s * RPT
        for k in range(RPT // CHK):
            pltpu.sync_copy(zv, outa.at[pl.ds(base + k * CHK, CHK)])
            pltpu.sync_copy(zv, ina.at[pl.ds(base + k * CHK, CHK)])
        plsc.subcore_barrier()

        pltpu.sync_copy(src_hbm.at[w], srcv)
        pltpu.sync_copy(dst_hbm.at[w], dstv)

        def body(j, carry):
            pltpu.sync_copy(onesv, outa.at[srcv.at[j]], add=True)
            pltpu.sync_copy(onesv, ina.at[dstv.at[j]], add=True)
            return carry
        lax.fori_loop(0, CPT, body, 0)
        plsc.subcore_barrier()

        pltpu.sync_copy(outa.at[pl.ds(base, RPT)], outd_hbm.at[c, pl.ds(base, RPT)])
        pltpu.sync_copy(ina.at[pl.ds(base, RPT)], ind_hbm.at[c, pl.ds(base, RPT)])

    return deg_kernel(src_r, dst_r)


def _sc_segsum(hp, src_r, dst_r, D):
    @functools.partial(
        pl.kernel,
        out_type=jax.ShapeDtypeStruct((2, NP, D), jnp.float32),
        mesh=_mesh(),
        scratch_types=[
            pltpu.VMEM((CPT, CHK), jnp.int32),
            pltpu.VMEM((CPT, CHK), jnp.int32),
            pltpu.VMEM((CHK, D), jnp.float32),
            pltpu.VMEM((CHK, D), jnp.float32),
            pltpu.VMEM((CHK, D), jnp.float32),
            pltpu.VMEM((CHK, D), jnp.float32),
            pltpu.VMEM((CHK, D), jnp.float32),
            pltpu.VMEM_SHARED((NP, D), jnp.float32),
            pltpu.SemaphoreType.DMA,
            pltpu.SemaphoreType.DMA,
            pltpu.SemaphoreType.DMA,
            pltpu.SemaphoreType.DMA,
            pltpu.SemaphoreType.DMA,
            pltpu.SemaphoreType.DMA,
            pltpu.SemaphoreType.DMA,
            pltpu.SemaphoreType.DMA,
        ],
        compiler_params=pltpu.CompilerParams(use_tc_tiling_on_sc=False),
    )
    def seg_kernel(hp_hbm, src_hbm, dst_hbm, out_hbm,
                   srcv, dstv, r0, r1, r2, r3, zv, acc,
                   g0, g1, g2, g3, s0, s1, s2, s3):
        c = lax.axis_index("c")
        s = lax.axis_index("s")
        w = c * 16 + s
        R = (r0, r1, r2, r3)
        GS = (g0, g1, g2, g3)
        SS = (s0, s1, s2, s3)

        def fill(i, carry):
            for q in range(D // 16):
                zv[i, pl.ds(q * 16, 16)] = jnp.zeros((16,), jnp.float32)
            return carry
        lax.fori_loop(0, CHK, fill, 0)

        base = s * RPT
        for k in range(RPT // CHK):
            pltpu.sync_copy(zv, acc.at[pl.ds(base + k * CHK, CHK)])
        plsc.subcore_barrier()

        pltpu.sync_copy(src_hbm.at[w], srcv)
        pltpu.sync_copy(dst_hbm.at[w], dstv)

        for b in range(4):
            pltpu.async_copy(hp_hbm.at[srcv.at[b]], R[b], GS[b])

        def body(i, carry):
            for b in range(4):
                j = 4 * i + b
                pltpu.make_async_copy(hp_hbm.at[srcv.at[j]], R[b], GS[b]).wait()
                pltpu.async_copy(R[b], acc.at[dstv.at[j]], SS[b], add=True)
            for b in range(4):
                j = 4 * i + b

                @pl.when(j + 4 < CPT)
                def _():
                    pltpu.make_async_copy(R[b], acc.at[dstv.at[j]], SS[b]).wait()
                    pltpu.async_copy(hp_hbm.at[srcv.at[j + 4]], R[b], GS[b])
            return carry
        lax.fori_loop(0, CPT // 4, body, 0)
        for b in range(4):
            pltpu.make_async_copy(R[b], acc.at[dstv.at[CPT - 4 + b]], SS[b]).wait()
        plsc.subcore_barrier()

        pltpu.sync_copy(acc.at[pl.ds(base, RPT)], out_hbm.at[c, pl.ds(base, RPT)])

    return seg_kernel(hp, src_r, dst_r)


def _topk_gru(feat, feat_ref, sct, Wu, Uu, bu, Wr, Ur, br, Wh, Uh, bh,
              Q, zref):
    f32 = jnp.float32
    M = feat.shape[0]
    nrm = jnp.maximum(jnp.sqrt(jnp.sum(sct * sct)), 1e-6)
    svec = lax.dot_general(sct, feat, (((1,), (1,)), ((), ())),
                           preferred_element_type=f32) / nrm
    lane = lax.broadcasted_iota(jnp.int32, (1, M), 1)
    if M > N:
        svec = jnp.where(lane < N, svec, -jnp.inf)
    row64 = lax.broadcasted_iota(jnp.int32, (H, 1), 0)
    vals = jnp.zeros((H, 1), f32)
    for i in range(H):
        m = jnp.max(svec)
        idx = jnp.min(jnp.where(svec == m, lane, M))
        svec = jnp.where(lane == idx, -jnp.inf, svec)
        vals = jnp.where(row64 == i, m, vals)
        zref[pl.ds(i, 1), :] = feat_ref[pl.ds(idx, 1), :]
    Zs = zref[...] * jnp.tanh(vals)

    def mmT(A, B):
        return lax.dot_general(A, B, (((1,), (1,)), ((), ())),
                               preferred_element_type=f32)

    u = jax.nn.sigmoid(mmT(Wu, Zs) + jnp.dot(Uu, Q, preferred_element_type=f32) + bu)
    r = jax.nn.sigmoid(mmT(Wr, Zs) + jnp.dot(Ur, Q, preferred_element_type=f32) + br)
    hc = jnp.tanh(mmT(Wh, Zs) + jnp.dot(Uh, r * Q, preferred_element_type=f32) + bh)
    return (1.0 - u) * Q + u * hc


def _rowmask(v):
    return jnp.where(lax.broadcasted_iota(jnp.int32, (NP, 1), 0) < N, v, 0.0)


def _tc_prep0(x, sct, Wu, Uu, bu, Wr, Ur, br, Wh, Uh, bh, Q, outd):
    def body(x_ref, sct_ref, Wu_r, Uu_r, bu_r, Wr_r, Ur_r, br_r,
             Wh_r, Uh_r, bh_r, Q_r, od_ref, We_ref, ha_ref, hb_ref, zref):
        We_ref[...] = _topk_gru(
            x_ref[...], x_ref, sct_ref[...],
            Wu_r[...], Uu_r[...], bu_r[...], Wr_r[...], Ur_r[...], br_r[...],
            Wh_r[...], Uh_r[...], bh_r[...], Q_r[...], zref)
        od = od_ref[0, 0:N, 0:1] + od_ref[1, 0:N, 0:1]
        routd = lax.rsqrt(jnp.maximum(od, 1.0))
        t = x_ref[...] * routd
        ha_ref[pl.ds(0, N), :] = t[:, :H]
        hb_ref[pl.ds(0, N), :] = t[:, H:]
        zpad = jnp.zeros((NP - N, H), jnp.float32)
        ha_ref[pl.ds(N, NP - N), :] = zpad
        hb_ref[pl.ds(N, NP - N), :] = zpad

    return pl.pallas_call(
        body,
        out_shape=[jax.ShapeDtypeStruct((IN, H), jnp.float32),
                   jax.ShapeDtypeStruct((NP, H), jnp.float32),
                   jax.ShapeDtypeStruct((NP, H), jnp.float32)],
        scratch_shapes=[pltpu.VMEM((H, IN), jnp.float32)],
    )(x, sct, Wu, Uu, bu, Wr, Ur, br, Wh, Uh, bh, Q, outd)


def _tc_mid(agga, aggb, We0, outd, ind, sct, Wu, Uu, bu, Wr, Ur, br,
            Wh, Uh, bh, Q):
    def body(agga_ref, aggb_ref, We0_r, od_ref, ind_ref, sct_ref, Wu_r,
             Uu_r, bu_r, Wr_r, Ur_r, br_r, Wh_r, Uh_r, bh_r, Q_r, h_ref,
             We_ref, featref, zref):
        ind_s = ind_ref[0, :, 0:1] + ind_ref[1, :, 0:1]
        rin = lax.rsqrt(jnp.maximum(ind_s, 1.0))
        agg = jnp.concatenate([agga_ref[0] + agga_ref[1],
                               aggb_ref[0] + aggb_ref[1]], axis=1)
        aggs = agg * rin
        t = jnp.dot(aggs, We0_r[...], preferred_element_type=jnp.float32)
        feat = jnp.where(t >= 0, t, SLOPE * t)
        featref[...] = feat
        od = od_ref[0, :, 0:1] + od_ref[1, :, 0:1]
        routd = lax.rsqrt(jnp.maximum(od, 1.0))
        We_ref[...] = _topk_gru(
            feat, featref, sct_ref[...],
            Wu_r[...], Uu_r[...], bu_r[...], Wr_r[...], Ur_r[...], br_r[...],
            Wh_r[...], Uh_r[...], bh_r[...], Q_r[...], zref)
        h_ref[...] = _rowmask(feat * routd)

    return pl.pallas_call(
        body,
        out_shape=[jax.ShapeDtypeStruct((NP, H), jnp.float32),
                   jax.ShapeDtypeStruct((H, H), jnp.float32)],
        scratch_shapes=[pltpu.VMEM((NP, H), jnp.float32),
                        pltpu.VMEM((H, H), jnp.float32)],
    )(agga, aggb, We0, outd, ind, sct, Wu, Uu, bu, Wr, Ur, br, Wh, Uh, bh, Q)


def _tc_mlp(agg, ind, We1, W1, b1, W2, b2):
    BR = 1280

    def body(agg_ref, ind_ref, We1_r, W1_r, b1_r, W2_r, b2_r, out_ref):
        ind_s = ind_ref[0, :, 0:1] + ind_ref[1, :, 0:1]
        rin = lax.rsqrt(jnp.maximum(ind_s, 1.0))
        aggs = (agg_ref[0] + agg_ref[1]) * rin
        t = jnp.dot(aggs, We1_r[...], preferred_element_type=jnp.float32)
        feat = jnp.where(t >= 0, t, SLOPE * t)
        hmid = jnp.maximum(
            jnp.dot(feat, W1_r[...], preferred_element_type=jnp.float32)
            + b1_r[...], 0.0)
        out_ref[...] = (jnp.dot(hmid, W2_r[...],
                                preferred_element_type=jnp.float32)
                        + b2_r[...])

    return pl.pallas_call(
        body,
        grid=(NP // BR,),
        in_specs=[
            pl.BlockSpec((2, BR, H), lambda i: (0, i, 0)),
            pl.BlockSpec((2, BR, 16), lambda i: (0, i, 0)),
            pl.BlockSpec((H, H), lambda i: (0, 0)),
            pl.BlockSpec((H, CH), lambda i: (0, 0)),
            pl.BlockSpec((1, CH), lambda i: (0, 0)),
            pl.BlockSpec((CH, NCLS), lambda i: (0, 0)),
            pl.BlockSpec((1, NCLS), lambda i: (0, 0)),
        ],
        out_specs=pl.BlockSpec((BR, NCLS), lambda i: (i, 0)),
        out_shape=jax.ShapeDtypeStruct((NP, NCLS), jnp.float32),
    )(agg, ind, We1, W1, b1, W2, b2)


def kernel(x, edge_index, scorer0, Wu0, Uu0, bu0, Wr0, Ur0, br0, Wh0, Uh0,
           bh0, gcnW0, scorer1, Wu1, Uu1, bu1, Wr1, Ur1, br1, Wh1, Uh1, bh1,
           gcnW1, mlpW1, mlpb1, mlpW2, mlpb2):
    src = edge_index[0].astype(jnp.int32)
    dst = edge_index[1].astype(jnp.int32)
    pad = N + jnp.arange(EPAD - E, dtype=jnp.int32) % (NP - N)
    src_r = jnp.concatenate([src, pad]).reshape(NTILES, CPT, CHK)
    dst_r = jnp.concatenate([dst, pad]).reshape(NTILES, CPT, CHK)

    outd, ind = _sc_degrees(src_r, dst_r)
    We0, h0a, h0b = _tc_prep0(x, scorer0.reshape(1, IN), Wu0, Uu0, bu0, Wr0,
                              Ur0, br0, Wh0, Uh0, bh0, gcnW0, outd)
    agg0a = _sc_segsum(h0a, src_r, dst_r, H)
    agg0b = _sc_segsum(h0b, src_r, dst_r, H)
    h1, We1 = _tc_mid(agg0a, agg0b, We0, outd, ind, scorer1.reshape(1, H),
                      Wu1, Uu1, bu1, Wr1, Ur1, br1, Wh1, Uh1, bh1, gcnW1)
    agg1 = _sc_segsum(h1, src_r, dst_r, H)
    out = _tc_mlp(agg1, ind, We1, mlpW1, mlpb1.reshape(1, CH), mlpW2,
                  mlpb2.reshape(1, NCLS))
    return out[:N]


# speedup vs baseline: 1.0847x; 1.0847x over previous
import functools

import jax
import jax.numpy as jnp
from jax import lax
from jax.experimental import pallas as pl
from jax.experimental.pallas import tpu as pltpu
from jax.experimental.pallas import tpu_sc as plsc

N = 10000
E = 320000
IN = 128
H = 64
CH = 512
NCLS = 2
SLOPE = 11.0 / 48.0

NP = 10240
NTILES = 32
CHK = 128
CPT = 80
EPAD = NTILES * CPT * CHK
RPT = NP // 16


def _mesh():
    return plsc.VectorSubcoreMesh(core_axis_name="c", subcore_axis_name="s")


def _sc_degrees(src_r, dst_r):
    @functools.partial(
        pl.kernel,
        out_type=[jax.ShapeDtypeStruct((2, NP, 16), jnp.float32),
                  jax.ShapeDtypeStruct((2, NP, 16), jnp.float32)],
        mesh=_mesh(),
        scratch_types=[
            pltpu.VMEM((CPT, CHK), jnp.int32),
            pltpu.VMEM((CPT, CHK), jnp.int32),
            pltpu.VMEM((CHK, 16), jnp.float32),
            pltpu.VMEM((CHK, 16), jnp.float32),
            pltpu.VMEM_SHARED((NP, 16), jnp.float32),
            pltpu.VMEM_SHARED((NP, 16), jnp.float32),
        ],
        compiler_params=pltpu.CompilerParams(use_tc_tiling_on_sc=False),
    )
    def deg_kernel(src_hbm, dst_hbm, outd_hbm, ind_hbm,
                   srcv, dstv, onesv, zv, outa, ina):
        c = lax.axis_index("c")
        s = lax.axis_index("s")
        w = c * 16 + s

        def fill(i, carry):
            onesv[i, :] = jnp.ones((16,), jnp.float32)
            zv[i, :] = jnp.zeros((16,), jnp.float32)
            return carry
        lax.fori_loop(0, CHK, fill, 0)

        base = s * RPT
        for k in range(RPT // CHK):
            pltpu.sync_copy(zv, outa.at[pl.ds(base + k * CHK, CHK)])
            pltpu.sync_copy(zv, ina.at[pl.ds(base + k * CHK, CHK)])
        plsc.subcore_barrier()

        pltpu.sync_copy(src_hbm.at[w], srcv)
        pltpu.sync_copy(dst_hbm.at[w], dstv)

        def body(j, carry):
            pltpu.sync_copy(onesv, outa.at[srcv.at[j]], add=True)
            pltpu.sync_copy(onesv, ina.at[dstv.at[j]], add=True)
            return carry
        lax.fori_loop(0, CPT, body, 0)
        plsc.subcore_barrier()

        pltpu.sync_copy(outa.at[pl.ds(base, RPT)], outd_hbm.at[c, pl.ds(base, RPT)])
        pltpu.sync_copy(ina.at[pl.ds(base, RPT)], ind_hbm.at[c, pl.ds(base, RPT)])

    return deg_kernel(src_r, dst_r)


def _sc_segsum(hps, src_r, dst_r):
    n_t = len(hps)

    @functools.partial(
        pl.kernel,
        out_type=[jax.ShapeDtypeStruct((2, NP, H), jnp.float32)
                  for _ in range(n_t)],
        mesh=_mesh(),
        scratch_types=[
            pltpu.VMEM((CPT, CHK), jnp.int32),
            pltpu.VMEM((CPT, CHK), jnp.int32),
            pltpu.VMEM((CHK, H), jnp.float32),
            pltpu.VMEM((CHK, H), jnp.float32),
            pltpu.VMEM((CHK, H), jnp.float32),
            pltpu.VMEM((CHK, H), jnp.float32),
            pltpu.VMEM((CHK, H), jnp.float32),
            pltpu.VMEM_SHARED((NP, H), jnp.float32),
            pltpu.SemaphoreType.DMA,
            pltpu.SemaphoreType.DMA,
            pltpu.SemaphoreType.DMA,
            pltpu.SemaphoreType.DMA,
            pltpu.SemaphoreType.DMA,
            pltpu.SemaphoreType.DMA,
            pltpu.SemaphoreType.DMA,
            pltpu.SemaphoreType.DMA,
        ],
        compiler_params=pltpu.CompilerParams(use_tc_tiling_on_sc=False),
    )
    def seg_kernel(*refs):
        hp_hbms = refs[:n_t]
        src_hbm, dst_hbm = refs[n_t], refs[n_t + 1]
        out_hbms = refs[n_t + 2:2 * n_t + 2]
        (srcv, dstv, r0, r1, r2, r3, zv, acc,
         g0, g1, g2, g3, s0, s1, s2, s3) = refs[2 * n_t + 2:]
        c = lax.axis_index("c")
        s = lax.axis_index("s")
        w = c * 16 + s
        R = (r0, r1, r2, r3)
        GS = (g0, g1, g2, g3)
        SS = (s0, s1, s2, s3)

        def fill(i, carry):
            for q in range(H // 16):
                zv[i, pl.ds(q * 16, 16)] = jnp.zeros((16,), jnp.float32)
            return carry
        lax.fori_loop(0, CHK, fill, 0)

        base = s * RPT
        pltpu.sync_copy(src_hbm.at[w], srcv)
        pltpu.sync_copy(dst_hbm.at[w], dstv)

        for t in range(n_t):
            hp_hbm = hp_hbms[t]
            for k in range(RPT // CHK):
                pltpu.sync_copy(zv, acc.at[pl.ds(base + k * CHK, CHK)])
            plsc.subcore_barrier()

            for b in range(4):
                pltpu.async_copy(hp_hbm.at[srcv.at[b]], R[b], GS[b])

            def body(i, carry):
                for b in range(4):
                    j = 4 * i + b
                    pltpu.make_async_copy(hp_hbm.at[srcv.at[j]], R[b], GS[b]).wait()
                    pltpu.async_copy(R[b], acc.at[dstv.at[j]], SS[b], add=True)
                for b in range(4):
                    j = 4 * i + b

                    @pl.when(j + 4 < CPT)
                    def _():
                        pltpu.make_async_copy(R[b], acc.at[dstv.at[j]], SS[b]).wait()
                        pltpu.async_copy(hp_hbm.at[srcv.at[j + 4]], R[b], GS[b])
                return carry
            lax.fori_loop(0, CPT // 4, body, 0)
            for b in range(4):
                pltpu.make_async_copy(R[b], acc.at[dstv.at[CPT - 4 + b]], SS[b]).wait()
            plsc.subcore_barrier()

            pltpu.sync_copy(acc.at[pl.ds(base, RPT)],
                            out_hbms[t].at[c, pl.ds(base, RPT)])

    return seg_kernel(*hps, src_r, dst_r)


def _topk_gru(feat, feat_ref, sct, Wu, Uu, bu, Wr, Ur, br, Wh, Uh, bh,
              Q, zref):
    f32 = jnp.float32
    M = feat.shape[0]
    nrm = jnp.maximum(jnp.sqrt(jnp.sum(sct * sct)), 1e-6)
    svec = lax.dot_general(sct, feat, (((1,), (1,)), ((), ())),
                           preferred_element_type=f32) / nrm
    lane = lax.broadcasted_iota(jnp.int32, (1, M), 1)
    if M > N:
        svec = jnp.where(lane < N, svec, -jnp.inf)
    row64 = lax.broadcasted_iota(jnp.int32, (H, 1), 0)
    vals = jnp.zeros((H, 1), f32)
    for i in range(H):
        m = jnp.max(svec)
        idx = jnp.min(jnp.where(svec == m, lane, M))
        svec = jnp.where(lane == idx, -jnp.inf, svec)
        vals = jnp.where(row64 == i, m, vals)
        zref[pl.ds(i, 1), :] = feat_ref[pl.ds(idx, 1), :]
    Zs = zref[...] * jnp.tanh(vals)

    def mmT(A, B):
        return lax.dot_general(A, B, (((1,), (1,)), ((), ())),
                               preferred_element_type=f32)

    u = jax.nn.sigmoid(mmT(Wu, Zs) + jnp.dot(Uu, Q, preferred_element_type=f32) + bu)
    r = jax.nn.sigmoid(mmT(Wr, Zs) + jnp.dot(Ur, Q, preferred_element_type=f32) + br)
    hc = jnp.tanh(mmT(Wh, Zs) + jnp.dot(Uh, r * Q, preferred_element_type=f32) + bh)
    return (1.0 - u) * Q + u * hc


def _rowmask(v):
    return jnp.where(lax.broadcasted_iota(jnp.int32, (NP, 1), 0) < N, v, 0.0)


def _tc_gru0(x, sct, Wu, Uu, bu, Wr, Ur, br, Wh, Uh, bh, Q):
    def body(x_ref, sct_ref, Wu_r, Uu_r, bu_r, Wr_r, Ur_r, br_r,
             Wh_r, Uh_r, bh_r, Q_r, We_ref, zref):
        We_ref[...] = _topk_gru(
            x_ref[...], x_ref, sct_ref[...],
            Wu_r[...], Uu_r[...], bu_r[...], Wr_r[...], Ur_r[...], br_r[...],
            Wh_r[...], Uh_r[...], bh_r[...], Q_r[...], zref)

    return pl.pallas_call(
        body,
        out_shape=jax.ShapeDtypeStruct((IN, H), jnp.float32),
        scratch_shapes=[pltpu.VMEM((H, IN), jnp.float32)],
    )(x, sct, Wu, Uu, bu, Wr, Ur, br, Wh, Uh, bh, Q)


def _tc_scale0(x, outd):
    def body(x_ref, od_ref, ha_ref, hb_ref):
        od = od_ref[0, 0:N, 0:1] + od_ref[1, 0:N, 0:1]
        routd = lax.rsqrt(jnp.maximum(od, 1.0))
        t = x_ref[...] * routd
        ha_ref[pl.ds(0, N), :] = t[:, :H]
        hb_ref[pl.ds(0, N), :] = t[:, H:]
        zpad = jnp.zeros((NP - N, H), jnp.float32)
        ha_ref[pl.ds(N, NP - N), :] = zpad
        hb_ref[pl.ds(N, NP - N), :] = zpad

    return pl.pallas_call(
        body,
        out_shape=[jax.ShapeDtypeStruct((NP, H), jnp.float32),
                   jax.ShapeDtypeStruct((NP, H), jnp.float32)],
    )(x, outd)


def _tc_mid(agga, aggb, We0, outd, ind, sct, Wu, Uu, bu, Wr, Ur, br,
            Wh, Uh, bh, Q):
    def body(agga_ref, aggb_ref, We0_r, od_ref, ind_ref, sct_ref, Wu_r,
             Uu_r, bu_r, Wr_r, Ur_r, br_r, Wh_r, Uh_r, bh_r, Q_r, h_ref,
             We_ref, featref, zref):
        ind_s = ind_ref[0, :, 0:1] + ind_ref[1, :, 0:1]
        rin = lax.rsqrt(jnp.maximum(ind_s, 1.0))
        agg = jnp.concatenate([agga_ref[0] + agga_ref[1],
                               aggb_ref[0] + aggb_ref[1]], axis=1)
        aggs = agg * rin
        t = jnp.dot(aggs, We0_r[...], preferred_element_type=jnp.float32)
        feat = jnp.where(t >= 0, t, SLOPE * t)
        featref[...] = feat
        od = od_ref[0, :, 0:1] + od_ref[1, :, 0:1]
        routd = lax.rsqrt(jnp.maximum(od, 1.0))
        We_ref[...] = _topk_gru(
            feat, featref, sct_ref[...],
            Wu_r[...], Uu_r[...], bu_r[...], Wr_r[...], Ur_r[...], br_r[...],
            Wh_r[...], Uh_r[...], bh_r[...], Q_r[...], zref)
        h_ref[...] = _rowmask(feat * routd)

    return pl.pallas_call(
        body,
        out_shape=[jax.ShapeDtypeStruct((NP, H), jnp.float32),
                   jax.ShapeDtypeStruct((H, H), jnp.float32)],
        scratch_shapes=[pltpu.VMEM((NP, H), jnp.float32),
                        pltpu.VMEM((H, H), jnp.float32)],
    )(agga, aggb, We0, outd, ind, sct, Wu, Uu, bu, Wr, Ur, br, Wh, Uh, bh, Q)


def _tc_mlp(agg, ind, We1, W1, b1, W2, b2):
    BR = 1280

    def body(agg_ref, ind_ref, We1_r, W1_r, b1_r, W2_r, b2_r, out_ref):
        ind_s = ind_ref[0, :, 0:1] + ind_ref[1, :, 0:1]
        rin = lax.rsqrt(jnp.maximum(ind_s, 1.0))
        aggs = (agg_ref[0] + agg_ref[1]) * rin
        t = jnp.dot(aggs, We1_r[...], preferred_element_type=jnp.float32)
        feat = jnp.where(t >= 0, t, SLOPE * t)
        hmid = jnp.maximum(
            jnp.dot(feat, W1_r[...], preferred_element_type=jnp.float32)
            + b1_r[...], 0.0)
        out_ref[...] = (jnp.dot(hmid, W2_r[...],
                                preferred_element_type=jnp.float32)
                        + b2_r[...])

    return pl.pallas_call(
        body,
        grid=(NP // BR,),
        in_specs=[
            pl.BlockSpec((2, BR, H), lambda i: (0, i, 0)),
            pl.BlockSpec((2, BR, 16), lambda i: (0, i, 0)),
            pl.BlockSpec((H, H), lambda i: (0, 0)),
            pl.BlockSpec((H, CH), lambda i: (0, 0)),
            pl.BlockSpec((1, CH), lambda i: (0, 0)),
            pl.BlockSpec((CH, NCLS), lambda i: (0, 0)),
            pl.BlockSpec((1, NCLS), lambda i: (0, 0)),
        ],
        out_specs=pl.BlockSpec((BR, NCLS), lambda i: (i, 0)),
        out_shape=jax.ShapeDtypeStruct((NP, NCLS), jnp.float32),
    )(agg, ind, We1, W1, b1, W2, b2)


def kernel(x, edge_index, scorer0, Wu0, Uu0, bu0, Wr0, Ur0, br0, Wh0, Uh0,
           bh0, gcnW0, scorer1, Wu1, Uu1, bu1, Wr1, Ur1, br1, Wh1, Uh1, bh1,
           gcnW1, mlpW1, mlpb1, mlpW2, mlpb2):
    src = edge_index[0].astype(jnp.int32)
    dst = edge_index[1].astype(jnp.int32)
    pad = N + jnp.arange(EPAD - E, dtype=jnp.int32) % (NP - N)
    src_r = jnp.concatenate([src, pad]).reshape(NTILES, CPT, CHK)
    dst_r = jnp.concatenate([dst, pad]).reshape(NTILES, CPT, CHK)

    outd, ind = _sc_degrees(src_r, dst_r)
    We0 = _tc_gru0(x, scorer0.reshape(1, IN), Wu0, Uu0, bu0, Wr0, Ur0,
                   br0, Wh0, Uh0, bh0, gcnW0)
    h0a, h0b = _tc_scale0(x, outd)
    agg0a, agg0b = _sc_segsum([h0a, h0b], src_r, dst_r)
    h1, We1 = _tc_mid(agg0a, agg0b, We0, outd, ind, scorer1.reshape(1, H),
                      Wu1, Uu1, bu1, Wr1, Ur1, br1, Wh1, Uh1, bh1, gcnW1)
    agg1, = _sc_segsum([h1], src_r, dst_r)
    out = _tc_mlp(agg1, ind, We1, mlpW1, mlpb1.reshape(1, CH), mlpW2,
                  mlpb2.reshape(1, NCLS))
    return out[:N]


# speedup vs baseline: 1.1050x; 1.0187x over previous
import functools

import jax
import jax.numpy as jnp
from jax import lax
from jax.experimental import pallas as pl
from jax.experimental.pallas import tpu as pltpu
from jax.experimental.pallas import tpu_sc as plsc

N = 10000
E = 320000
IN = 128
H = 64
CH = 512
NCLS = 2
SLOPE = 11.0 / 48.0

NP = 10240
NTILES = 32
CHK = 128
CPT = 80
EPAD = NTILES * CPT * CHK
RPT = NP // 16


def _mesh():
    return plsc.VectorSubcoreMesh(core_axis_name="c", subcore_axis_name="s")


def _sc_degrees(src_r, dst_r):
    @functools.partial(
        pl.kernel,
        out_type=[jax.ShapeDtypeStruct((2, NP, 16), jnp.float32),
                  jax.ShapeDtypeStruct((2, NP, 16), jnp.float32)],
        mesh=_mesh(),
        scratch_types=[
            pltpu.VMEM((CPT, CHK), jnp.int32),
            pltpu.VMEM((CPT, CHK), jnp.int32),
            pltpu.VMEM((CHK, 16), jnp.float32),
            pltpu.VMEM((CHK, 16), jnp.float32),
            pltpu.VMEM_SHARED((NP, 16), jnp.float32),
            pltpu.VMEM_SHARED((NP, 16), jnp.float32),
        ],
        compiler_params=pltpu.CompilerParams(use_tc_tiling_on_sc=False),
    )
    def deg_kernel(src_hbm, dst_hbm, outd_hbm, ind_hbm,
                   srcv, dstv, onesv, zv, outa, ina):
        c = lax.axis_index("c")
        s = lax.axis_index("s")
        w = c * 16 + s

        def fill(i, carry):
            onesv[i, :] = jnp.ones((16,), jnp.float32)
            zv[i, :] = jnp.zeros((16,), jnp.float32)
            return carry
        lax.fori_loop(0, CHK, fill, 0)

        base = s * RPT
        for k in range(RPT // CHK):
            pltpu.sync_copy(zv, outa.at[pl.ds(base + k * CHK, CHK)])
            pltpu.sync_copy(zv, ina.at[pl.ds(base + k * CHK, CHK)])
        plsc.subcore_barrier()

        pltpu.sync_copy(src_hbm.at[w], srcv)
        pltpu.sync_copy(dst_hbm.at[w], dstv)

        def body(j, carry):
            pltpu.sync_copy(onesv, outa.at[srcv.at[j]], add=True)
            pltpu.sync_copy(onesv, ina.at[dstv.at[j]], add=True)
            return carry
        lax.fori_loop(0, CPT, body, 0)
        plsc.subcore_barrier()

        pltpu.sync_copy(outa.at[pl.ds(base, RPT)], outd_hbm.at[c, pl.ds(base, RPT)])
        pltpu.sync_copy(ina.at[pl.ds(base, RPT)], ind_hbm.at[c, pl.ds(base, RPT)])

    return deg_kernel(src_r, dst_r)


def _sc_segsum(hps, src_r, dst_r):
    n_t = len(hps)

    @functools.partial(
        pl.kernel,
        out_type=[jax.ShapeDtypeStruct((2, NP, H), jnp.float32)
                  for _ in range(n_t)],
        mesh=_mesh(),
        scratch_types=[
            pltpu.VMEM((CPT, CHK), jnp.int32),
            pltpu.VMEM((CPT, CHK), jnp.int32),
            pltpu.VMEM((CHK, H), jnp.float32),
            pltpu.VMEM((CHK, H), jnp.float32),
            pltpu.VMEM((CHK, H), jnp.float32),
            pltpu.VMEM((CHK, H), jnp.float32),
            pltpu.VMEM((CHK, H), jnp.float32),
            pltpu.VMEM_SHARED((NP, H), jnp.float32),
            pltpu.SemaphoreType.DMA,
            pltpu.SemaphoreType.DMA,
            pltpu.SemaphoreType.DMA,
            pltpu.SemaphoreType.DMA,
            pltpu.SemaphoreType.DMA,
            pltpu.SemaphoreType.DMA,
            pltpu.SemaphoreType.DMA,
            pltpu.SemaphoreType.DMA,
        ],
        compiler_params=pltpu.CompilerParams(use_tc_tiling_on_sc=False),
    )
    def seg_kernel(*refs):
        hp_hbms = refs[:n_t]
        src_hbm, dst_hbm = refs[n_t], refs[n_t + 1]
        out_hbms = refs[n_t + 2:2 * n_t + 2]
        (srcv, dstv, r0, r1, r2, r3, zv, acc,
         g0, g1, g2, g3, s0, s1, s2, s3) = refs[2 * n_t + 2:]
        c = lax.axis_index("c")
        s = lax.axis_index("s")
        w = c * 16 + s
        R = (r0, r1, r2, r3)
        GS = (g0, g1, g2, g3)
        SS = (s0, s1, s2, s3)

        def fill(i, carry):
            for q in range(H // 16):
                zv[i, pl.ds(q * 16, 16)] = jnp.zeros((16,), jnp.float32)
            return carry
        lax.fori_loop(0, CHK, fill, 0)

        base = s * RPT
        pltpu.sync_copy(src_hbm.at[w], srcv)
        pltpu.sync_copy(dst_hbm.at[w], dstv)

        for t in range(n_t):
            hp_hbm = hp_hbms[t]
            for k in range(RPT // CHK):
                pltpu.sync_copy(zv, acc.at[pl.ds(base + k * CHK, CHK)])
            plsc.subcore_barrier()

            for b in range(4):
                pltpu.async_copy(hp_hbm.at[srcv.at[b]], R[b], GS[b])

            def body(i, carry):
                for b in range(4):
                    j = 4 * i + b
                    pltpu.make_async_copy(hp_hbm.at[srcv.at[j]], R[b], GS[b]).wait()
                    pltpu.async_copy(R[b], acc.at[dstv.at[j]], SS[b], add=True)
                for b in range(4):
                    j = 4 * i + b

                    @pl.when(j + 4 < CPT)
                    def _():
                        pltpu.make_async_copy(R[b], acc.at[dstv.at[j]], SS[b]).wait()
                        pltpu.async_copy(hp_hbm.at[srcv.at[j + 4]], R[b], GS[b])
                return carry
            lax.fori_loop(0, CPT // 4, body, 0)
            for b in range(4):
                pltpu.make_async_copy(R[b], acc.at[dstv.at[CPT - 4 + b]], SS[b]).wait()
            plsc.subcore_barrier()

            pltpu.sync_copy(acc.at[pl.ds(base, RPT)],
                            out_hbms[t].at[c, pl.ds(base, RPT)])

    return seg_kernel(*hps, src_r, dst_r)


def _topk_gru(feat, feat_ref, sct, Wu, Uu, bu, Wr, Ur, br, Wh, Uh, bh,
              Q, zref, sref):
    f32 = jnp.float32
    M = feat.shape[0]
    nrm = jnp.maximum(jnp.sqrt(jnp.sum(sct * sct)), 1e-6)
    svec = lax.dot_general(sct, feat, (((1,), (1,)), ((), ())),
                           preferred_element_type=f32) / nrm
    lane = lax.broadcasted_iota(jnp.int32, (1, M), 1)
    if M > N:
        svec = jnp.where(lane < N, svec, -jnp.inf)
    if M < NP:
        svec = jnp.concatenate(
            [svec, jnp.full((1, NP - M), -jnp.inf, f32)], axis=1)
    for k in range(NP // 128):
        sref[pl.ds(k, 1), :] = svec[:, k * 128:(k + 1) * 128]
    s2d = sref[...]
    gidx = (lax.broadcasted_iota(jnp.int32, (NP // 128, 128), 0) * 128
            + lax.broadcasted_iota(jnp.int32, (NP // 128, 128), 1))
    row64 = lax.broadcasted_iota(jnp.int32, (H, 1), 0)
    vals = jnp.zeros((H, 1), f32)
    for i in range(H):
        m = jnp.max(s2d)
        g = jnp.min(jnp.where(s2d == m, gidx, NP))
        s2d = jnp.where(gidx == g, -jnp.inf, s2d)
        vals = jnp.where(row64 == i, m, vals)
        zref[pl.ds(i, 1), :] = feat_ref[pl.ds(g, 1), :]
    Zs = zref[...] * jnp.tanh(vals)

    def mmT(A, B):
        return lax.dot_general(A, B, (((1,), (1,)), ((), ())),
                               preferred_element_type=f32)

    u = jax.nn.sigmoid(mmT(Wu, Zs) + jnp.dot(Uu, Q, preferred_element_type=f32) + bu)
    r = jax.nn.sigmoid(mmT(Wr, Zs) + jnp.dot(Ur, Q, preferred_element_type=f32) + br)
    hc = jnp.tanh(mmT(Wh, Zs) + jnp.dot(Uh, r * Q, preferred_element_type=f32) + bh)
    return (1.0 - u) * Q + u * hc


def _rowmask(v):
    return jnp.where(lax.broadcasted_iota(jnp.int32, (NP, 1), 0) < N, v, 0.0)


def _tc_gru0(x, sct, Wu, Uu, bu, Wr, Ur, br, Wh, Uh, bh, Q):
    def body(x_ref, sct_ref, Wu_r, Uu_r, bu_r, Wr_r, Ur_r, br_r,
             Wh_r, Uh_r, bh_r, Q_r, We_ref, zref, sref):
        We_ref[...] = _topk_gru(
            x_ref[...], x_ref, sct_ref[...],
            Wu_r[...], Uu_r[...], bu_r[...], Wr_r[...], Ur_r[...], br_r[...],
            Wh_r[...], Uh_r[...], bh_r[...], Q_r[...], zref, sref)

    return pl.pallas_call(
        body,
        out_shape=jax.ShapeDtypeStruct((IN, H), jnp.float32),
        scratch_shapes=[pltpu.VMEM((H, IN), jnp.float32),
                        pltpu.VMEM((NP // 128, 128), jnp.float32)],
    )(x, sct, Wu, Uu, bu, Wr, Ur, br, Wh, Uh, bh, Q)


def _tc_scale0(x, outd):
    def body(x_ref, od_ref, ha_ref, hb_ref):
        od = od_ref[0, 0:N, 0:1] + od_ref[1, 0:N, 0:1]
        routd = lax.rsqrt(jnp.maximum(od, 1.0))
        t = x_ref[...] * routd
        ha_ref[pl.ds(0, N), :] = t[:, :H]
        hb_ref[pl.ds(0, N), :] = t[:, H:]
        zpad = jnp.zeros((NP - N, H), jnp.float32)
        ha_ref[pl.ds(N, NP - N), :] = zpad
        hb_ref[pl.ds(N, NP - N), :] = zpad

    return pl.pallas_call(
        body,
        out_shape=[jax.ShapeDtypeStruct((NP, H), jnp.float32),
                   jax.ShapeDtypeStruct((NP, H), jnp.float32)],
    )(x, outd)


def _tc_mid(agga, aggb, We0, outd, ind, sct, Wu, Uu, bu, Wr, Ur, br,
            Wh, Uh, bh, Q):
    def body(agga_ref, aggb_ref, We0_r, od_ref, ind_ref, sct_ref, Wu_r,
             Uu_r, bu_r, Wr_r, Ur_r, br_r, Wh_r, Uh_r, bh_r, Q_r, h_ref,
             We_ref, featref, zref, sref):
        ind_s = ind_ref[0, :, 0:1] + ind_ref[1, :, 0:1]
        rin = lax.rsqrt(jnp.maximum(ind_s, 1.0))
        agg = jnp.concatenate([agga_ref[0] + agga_ref[1],
                               aggb_ref[0] + aggb_ref[1]], axis=1)
        aggs = agg * rin
        t = jnp.dot(aggs, We0_r[...], preferred_element_type=jnp.float32)
        feat = jnp.where(t >= 0, t, SLOPE * t)
        featref[...] = feat
        od = od_ref[0, :, 0:1] + od_ref[1, :, 0:1]
        routd = lax.rsqrt(jnp.maximum(od, 1.0))
        We_ref[...] = _topk_gru(
            feat, featref, sct_ref[...],
            Wu_r[...], Uu_r[...], bu_r[...], Wr_r[...], Ur_r[...], br_r[...],
            Wh_r[...], Uh_r[...], bh_r[...], Q_r[...], zref, sref)
        h_ref[...] = _rowmask(feat * routd)

    return pl.pallas_call(
        body,
        out_shape=[jax.ShapeDtypeStruct((NP, H), jnp.float32),
                   jax.ShapeDtypeStruct((H, H), jnp.float32)],
        scratch_shapes=[pltpu.VMEM((NP, H), jnp.float32),
                        pltpu.VMEM((H, H), jnp.float32),
                        pltpu.VMEM((NP // 128, 128), jnp.float32)],
    )(agga, aggb, We0, outd, ind, sct, Wu, Uu, bu, Wr, Ur, br, Wh, Uh, bh, Q)


def _tc_mlp(agg, ind, We1, W1, b1, W2, b2):
    BR = 1280

    def body(agg_ref, ind_ref, We1_r, W1_r, b1_r, W2_r, b2_r, out_ref):
        ind_s = ind_ref[0, :, 0:1] + ind_ref[1, :, 0:1]
        rin = lax.rsqrt(jnp.maximum(ind_s, 1.0))
        aggs = (agg_ref[0] + agg_ref[1]) * rin
        t = jnp.dot(aggs, We1_r[...], preferred_element_type=jnp.float32)
        feat = jnp.where(t >= 0, t, SLOPE * t)
        hmid = jnp.maximum(
            jnp.dot(feat, W1_r[...], preferred_element_type=jnp.float32)
            + b1_r[...], 0.0)
        out_ref[...] = (jnp.dot(hmid, W2_r[...],
                                preferred_element_type=jnp.float32)
                        + b2_r[...])

    return pl.pallas_call(
        body,
        grid=(NP // BR,),
        in_specs=[
            pl.BlockSpec((2, BR, H), lambda i: (0, i, 0)),
            pl.BlockSpec((2, BR, 16), lambda i: (0, i, 0)),
            pl.BlockSpec((H, H), lambda i: (0, 0)),
            pl.BlockSpec((H, CH), lambda i: (0, 0)),
            pl.BlockSpec((1, CH), lambda i: (0, 0)),
            pl.BlockSpec((CH, NCLS), lambda i: (0, 0)),
            pl.BlockSpec((1, NCLS), lambda i: (0, 0)),
        ],
        out_specs=pl.BlockSpec((BR, NCLS), lambda i: (i, 0)),
        out_shape=jax.ShapeDtypeStruct((NP, NCLS), jnp.float32),
    )(agg, ind, We1, W1, b1, W2, b2)


def kernel(x, edge_index, scorer0, Wu0, Uu0, bu0, Wr0, Ur0, br0, Wh0, Uh0,
           bh0, gcnW0, scorer1, Wu1, Uu1, bu1, Wr1, Ur1, br1, Wh1, Uh1, bh1,
           gcnW1, mlpW1, mlpb1, mlpW2, mlpb2):
    src = edge_index[0].astype(jnp.int32)
    dst = edge_index[1].astype(jnp.int32)
    pad = N + jnp.arange(EPAD - E, dtype=jnp.int32) % (NP - N)
    src_r = jnp.concatenate([src, pad]).reshape(NTILES, CPT, CHK)
    dst_r = jnp.concatenate([dst, pad]).reshape(NTILES, CPT, CHK)

    outd, ind = _sc_degrees(src_r, dst_r)
    We0 = _tc_gru0(x, scorer0.reshape(1, IN), Wu0, Uu0, bu0, Wr0, Ur0,
                   br0, Wh0, Uh0, bh0, gcnW0)
    h0a, h0b = _tc_scale0(x, outd)
    agg0a, = _sc_segsum([h0a], src_r, dst_r)
    agg0b, = _sc_segsum([h0b], src_r, dst_r)
    h1, We1 = _tc_mid(agg0a, agg0b, We0, outd, ind, scorer1.reshape(1, H),
                      Wu1, Uu1, bu1, Wr1, Ur1, br1, Wh1, Uh1, bh1, gcnW1)
    agg1, = _sc_segsum([h1], src_r, dst_r)
    out = _tc_mlp(agg1, ind, We1, mlpW1, mlpb1.reshape(1, CH), mlpW2,
                  mlpb2.reshape(1, NCLS))
    return out[:N]
